# baseline (speedup 1.0000x reference)
import functools
import math

import jax
import jax.numpy as jnp
from jax import lax
from jax.experimental import pallas as pl
from jax.experimental.pallas import tpu as pltpu
from jax.experimental.pallas import tpu_sc as plsc

VOCAB = 1000000
D_MODEL = 64
SCALE = math.sqrt(D_MODEL)

NC = 2
NS = 16
NW = NC * NS

BLK = 128
NBUF = 4


def _sc_embed(x3, lut):
    nblk = x3.shape[1]
    mesh = plsc.VectorSubcoreMesh(core_axis_name="c", subcore_axis_name="s")

    @functools.partial(
        pl.kernel,
        mesh=mesh,
        out_type=jax.ShapeDtypeStruct((NW, nblk, BLK, D_MODEL), jnp.float32),
        scratch_types=[
            pltpu.VMEM((nblk, BLK), jnp.int32),
            pltpu.VMEM((NBUF, BLK, D_MODEL), jnp.float32),
            pltpu.VMEM((NBUF, BLK, D_MODEL), jnp.float32),
            pltpu.SemaphoreType.DMA((NBUF,)),
            pltpu.SemaphoreType.DMA((NBUF,)),
        ],
        compiler_params=pltpu.CompilerParams(use_tc_tiling_on_sc=False),
    )
    def k(x_hbm, lut_hbm, out_hbm, idx_v, inbuf, outbuf, gsem, ssem):
        wid = lax.axis_index("s") * NC + lax.axis_index("c")

        pltpu.sync_copy(x_hbm.at[wid], idx_v)

        def fire_gather(j, b):
            pltpu.async_copy(lut_hbm.at[idx_v.at[j]], inbuf.at[b], gsem.at[b])

        def wait_gather(j, b):
            pltpu.make_async_copy(
                lut_hbm.at[idx_v.at[j]], inbuf.at[b], gsem.at[b]
            ).wait()

        def fire_store(j, b):
            pltpu.async_copy(outbuf.at[b], out_hbm.at[wid, j], ssem.at[b])

        def wait_store(j, b):
            pltpu.make_async_copy(
                outbuf.at[b], out_hbm.at[wid, j], ssem.at[b]
            ).wait()

        for b in range(NBUF):
            fire_gather(b, b)

        def outer(jbase, carry):
            for b in range(NBUF):
                j = jbase + b
                wait_gather(j, b)

                @pl.when(j >= NBUF)
                def _():
                    wait_store(j - NBUF, b)

                def scale_row(r, c):
                    for q in range(D_MODEL // 16):
                        outbuf[b, r, pl.ds(16 * q, 16)] = (
                            inbuf[b, r, pl.ds(16 * q, 16)] * SCALE
                        )
                    return c

                lax.fori_loop(0, BLK, scale_row, 0)

                @pl.when(j + NBUF < nblk)
                def _():
                    fire_gather(j + NBUF, b)

                fire_store(j, b)
            return carry

        lax.fori_loop(0, nblk // NBUF, lambda i, c: outer(i * NBUF, c), 0)

        for b in range(NBUF):
            wait_store(nblk - NBUF + b, b)

    return k(x3, lut)


def kernel(x, lut):
    n = x.shape[0] * x.shape[1]
    per_w = n // NW
    nblk = per_w // BLK
    x3 = x.reshape(NW, nblk, BLK).astype(jnp.int32)
    out = _sc_embed(x3, lut)
    return out.reshape(x.shape[0], x.shape[1], D_MODEL)
